# baseline (speedup 1.0000x reference)
import functools

import jax
import jax.numpy as jnp
from jax.experimental import pallas as pl
from jax.experimental.pallas import tpu as pltpu

N = 2048
_F32 = jnp.float32


def _spmm_group(groups, bm=256, bf16=False):
    adjs = [g[0] for g in groups]
    arr_to_idx = {}
    arrs = []

    def _arr_idx(a):
        if id(a) not in arr_to_idx:
            arr_to_idx[id(a)] = len(arrs)
            arrs.append(a)
        return arr_to_idx[id(a)]

    p_key_to_idx = {}
    p_specs = []
    items = []
    for ai, (_, its) in enumerate(groups):
        for (x, w, act) in its:
            key = (id(x), id(w))
            if key not in p_key_to_idx:
                p_key_to_idx[key] = len(p_specs)
                p_specs.append((_arr_idx(x), _arr_idx(w), w.shape[1]))
            items.append((ai, p_key_to_idx[key], act, w.shape[1]))

    n_adj = len(adjs)
    n_arr = len(arrs)
    n_out = len(items)

    def kfn(*refs):
        adj_refs = refs[:n_adj]
        arr_refs = refs[n_adj:n_adj + n_arr]
        out_refs = refs[n_adj + n_arr:n_adj + n_arr + n_out]
        p_refs = refs[n_adj + n_arr + n_out:]
        i = pl.program_id(0)

        @pl.when(i == 0)
        def _compute_panels():
            for u, (xi, wi, _) in enumerate(p_specs):
                p = jnp.dot(arr_refs[xi][...], arr_refs[wi][...],
                            preferred_element_type=_F32)
                p_refs[u][...] = p.astype(jnp.bfloat16) if bf16 else p

        for t, (ai, pi, act, _) in enumerate(items):
            y = jnp.dot(adj_refs[ai][...], p_refs[pi][...],
                        preferred_element_type=_F32)
            out_refs[t][...] = jnp.maximum(y, 0.0) if act else y

    in_specs = []
    operands = []
    for a in adjs:
        in_specs.append(pl.BlockSpec((bm, N), lambda i: (i, 0)))
        operands.append(a)
    for a in arrs:
        in_specs.append(pl.BlockSpec(a.shape, lambda i: (0, 0)))
        operands.append(a)

    out_specs = [pl.BlockSpec((bm, wd), lambda i: (i, 0))
                 for (_, _, _, wd) in items]
    out_shape = [jax.ShapeDtypeStruct((N, wd), _F32)
                 for (_, _, _, wd) in items]
    p_dtype = jnp.bfloat16 if bf16 else _F32
    scratch_shapes = [pltpu.VMEM((N, wd), p_dtype) for (_, _, wd) in p_specs]

    outs = pl.pallas_call(
        kfn,
        grid=(N // bm,),
        in_specs=in_specs,
        out_specs=out_specs,
        out_shape=out_shape,
        scratch_shapes=scratch_shapes,
    )(*operands)
    return list(outs)


def _att_pair(e1, e2, w_ref, u_ref):
    u_row = jnp.reshape(u_ref[...], (1, -1))
    v1 = jnp.tanh(jnp.dot(e1, w_ref[...], preferred_element_type=_F32))
    v2 = jnp.tanh(jnp.dot(e2, w_ref[...], preferred_element_type=_F32))
    t1 = jnp.sum(v1 * u_row, axis=1, keepdims=True)
    t2 = jnp.sum(v2 * u_row, axis=1, keepdims=True)
    m = jnp.maximum(t1, t2)
    z1 = jnp.exp(t1 - m)
    z2 = jnp.exp(t2 - m)
    s = z1 + z2
    a1 = z1 / s
    a2 = z2 / s
    comb = a1 * e1 + a2 * e2
    alpha = jnp.concatenate([a1, a2], axis=1)
    return comb, alpha


def _att_kernel(spa_ref, ls1_ref, fea_ref, lf1_ref, ls2_ref, lf2_ref,
                w1_ref, u1_ref, w2_ref, u2_ref, wf_ref, uf_ref,
                wo2_ref, uo2_ref, wc_ref, uc_ref,
                o1_ref, o2_ref, d_ref, a1_ref, a2_ref, af_ref, ao2_ref,
                ac_ref):
    att1, alpha1 = _att_pair(spa_ref[...], ls1_ref[...], w1_ref, u1_ref)
    att2, alpha2 = _att_pair(fea_ref[...], lf1_ref[...], w2_ref, u2_ref)
    o1, alphaf = _att_pair(att1, att2, wf_ref, uf_ref)
    o2, alphao2 = _att_pair(ls2_ref[...], lf2_ref[...], wo2_ref, uo2_ref)
    d, alphac = _att_pair(o1, o2, wc_ref, uc_ref)
    o1_ref[...] = o1
    o2_ref[...] = o2
    d_ref[...] = d
    a1_ref[...] = alpha1
    a2_ref[...] = alpha2
    af_ref[...] = alphaf
    ao2_ref[...] = alphao2
    ac_ref[...] = alphac


def _attention(emb_spa, latent_spa1, emb_fea, latent_fea1, latent_spa2,
               latent_fea2, att1_w, att1_u, att2_w, att2_u, attf_w, attf_u,
               atto2_w, atto2_u, attc_w, attc_u):
    d = emb_spa.shape[1]
    out_shape = [
        jax.ShapeDtypeStruct((N, d), _F32),
        jax.ShapeDtypeStruct((N, d), _F32),
        jax.ShapeDtypeStruct((N, d), _F32),
        jax.ShapeDtypeStruct((N, 2), _F32),
        jax.ShapeDtypeStruct((N, 2), _F32),
        jax.ShapeDtypeStruct((N, 2), _F32),
        jax.ShapeDtypeStruct((N, 2), _F32),
        jax.ShapeDtypeStruct((N, 2), _F32),
    ]
    return pl.pallas_call(_att_kernel, out_shape=out_shape)(
        emb_spa, latent_spa1, emb_fea, latent_fea1, latent_spa2, latent_fea2,
        att1_w, att1_u, att2_w, att2_u, attf_w, attf_u, atto2_w, atto2_u,
        attc_w, attc_u)


def kernel(f_omics1, f_omics2, adj_spa1, adj_fea1, adj_spa2, adj_fea2,
           cell_emb, adj_emb, W_emb_enc, W_emb_dec,
           enc1_W1, enc1_W2, enc1_W3, dec1_W1, dec1_W2, dec1_W3,
           enc2_W1, enc2_W2, enc2_W3, dec2_W1, dec2_W2, dec2_W3,
           att1_w, att1_u, att2_w, att2_u, attf_w, attf_u,
           atto2_w, atto2_u, attc_w, attc_u):
    spa1_16 = adj_spa1.astype(jnp.bfloat16)
    spa2_16 = adj_spa2.astype(jnp.bfloat16)
    emb_16 = adj_emb.astype(jnp.bfloat16)
    (emb_spa, s1x1, emb_fea, s2x1, g1x1, g2x1) = _spmm_group([
        (adj_spa1, [(cell_emb, W_emb_enc, False), (f_omics1, enc1_W1, True)]),
        (adj_emb, [(cell_emb, W_emb_enc, False)]),
        (adj_spa2, [(f_omics2, enc2_W1, True)]),
        (adj_fea1, [(f_omics1, enc1_W1, True)]),
        (adj_fea2, [(f_omics2, enc2_W1, True)]),
    ])
    (s1x2, s2x2, g1x2, g2x2) = _spmm_group([
        (adj_spa1, [(s1x1, enc1_W2, True)]),
        (adj_spa2, [(s2x1, enc2_W2, True)]),
        (adj_fea1, [(g1x1, enc1_W2, True)]),
        (adj_fea2, [(g2x1, enc2_W2, True)]),
    ])
    (latent_spa1, latent_spa2, latent_fea1, latent_fea2) = _spmm_group([
        (adj_spa1, [(s1x2, enc1_W3, False)]),
        (adj_spa2, [(s2x2, enc2_W3, False)]),
        (adj_fea1, [(g1x2, enc1_W3, False)]),
        (adj_fea2, [(g2x2, enc2_W3, False)]),
    ])
    (o1, o2, comb, alpha_att1, alpha_att2, alpha_att_omics1, alpha_omics2,
     alpha) = _attention(
        emb_spa, latent_spa1, emb_fea, latent_fea1, latent_spa2, latent_fea2,
        att1_w, att1_u, att2_w, att2_u, attf_w, attf_u, atto2_w, atto2_u,
        attc_w, attc_u)
    (r1_1, c2_1, recon_spa) = _spmm_group([
        (spa1_16, [(comb, dec1_W1, True), (o2, dec1_W1, True),
                   (emb_spa, W_emb_dec, False)]),
    ], bf16=True)
    (r2_1, c1_1, recon_fea) = _spmm_group([
        (spa2_16, [(comb, dec2_W1, True), (o1, dec2_W1, True)]),
        (emb_16, [(emb_fea, W_emb_dec, False)]),
    ], bf16=True)
    (r1_2, c2_2) = _spmm_group([
        (spa1_16, [(r1_1, dec1_W2, True), (c2_1, dec1_W2, True)]),
    ], bf16=True)
    (r2_2, c1_2) = _spmm_group([
        (spa2_16, [(r2_1, dec2_W2, True), (c1_1, dec2_W2, True)]),
    ], bf16=True)
    (emb_recon1, c2_3) = _spmm_group([
        (spa1_16, [(r1_2, dec1_W3, False), (c2_2, dec1_W3, False)]),
    ], bf16=True)
    (emb_recon2, c1_3) = _spmm_group([
        (spa2_16, [(r2_2, dec2_W3, False), (c1_2, dec2_W3, False)]),
    ], bf16=True)
    (e2x1, e1x1) = _spmm_group([
        (spa1_16, [(c2_3, enc1_W1, True)]),
        (spa2_16, [(c1_3, enc2_W1, True)]),
    ], bf16=True)
    (e2x2, e1x2) = _spmm_group([
        (spa1_16, [(e2x1, enc1_W2, True)]),
        (spa2_16, [(e1x1, enc2_W2, True)]),
    ], bf16=True)
    (emb_cross2, emb_cross1) = _spmm_group([
        (spa1_16, [(e2x2, enc1_W3, False)]),
        (spa2_16, [(e1x2, enc2_W3, False)]),
    ], bf16=True)
    return (o1, o2, comb, emb_recon1, emb_recon2, emb_cross1, emb_cross2,
            alpha_att1, alpha_att2, alpha_att_omics1, alpha_omics2, alpha,
            recon_spa, recon_fea)


# speedup vs baseline: 1.0797x; 1.0797x over previous
import jax
import jax.numpy as jnp
from jax.experimental import pallas as pl

N = 2048
_F32 = jnp.float32
_BF16 = jnp.bfloat16


def _dot(a, b):
    return jnp.dot(a, b, preferred_element_type=_F32)


def _relu(x):
    return jnp.maximum(x, 0.0)


def _enc_spa1_kernel(adj_ref, cell_ref, wenc_ref, wdec_ref, f_ref,
                     w1_ref, w2_ref, w3_ref,
                     emb_ref, lat_ref, recon_ref):
    a = adj_ref[...]
    c = _dot(cell_ref[...], wenc_ref[...])
    emb = _dot(a, c)
    emb_ref[...] = emb
    recon_ref[...] = _dot(a, _dot(emb, wdec_ref[...]))
    x = _relu(_dot(a, _dot(f_ref[...], w1_ref[...])))
    x = _relu(_dot(a, _dot(x, w2_ref[...])))
    lat_ref[...] = _dot(a, _dot(x, w3_ref[...]))


def _enc_spa1(adj, cell, wenc, wdec, f, w1, w2, w3):
    out_shape = [
        jax.ShapeDtypeStruct((N, wenc.shape[1]), _F32),
        jax.ShapeDtypeStruct((N, w3.shape[1]), _F32),
        jax.ShapeDtypeStruct((N, wdec.shape[1]), _F32),
    ]
    return pl.pallas_call(_enc_spa1_kernel, out_shape=out_shape)(
        adj, cell, wenc, wdec, f, w1, w2, w3)


def _enc_plain_kernel(adj_ref, f_ref, w1_ref, w2_ref, w3_ref, lat_ref):
    a = adj_ref[...]
    x = _relu(_dot(a, _dot(f_ref[...], w1_ref[...])))
    x = _relu(_dot(a, _dot(x, w2_ref[...])))
    lat_ref[...] = _dot(a, _dot(x, w3_ref[...]))


def _enc_plain(adj, f, w1, w2, w3):
    out_shape = jax.ShapeDtypeStruct((N, w3.shape[1]), _F32)
    return pl.pallas_call(_enc_plain_kernel, out_shape=out_shape)(
        adj, f, w1, w2, w3)


def _emb_kernel(adj_ref, cell_ref, wenc_ref, wdec_ref, emb_ref, recon_ref):
    a = adj_ref[...]
    emb = _dot(a, _dot(cell_ref[...], wenc_ref[...]))
    emb_ref[...] = emb
    recon_ref[...] = _dot(a, _dot(emb, wdec_ref[...]))


def _emb_chain(adj, cell, wenc, wdec):
    out_shape = [
        jax.ShapeDtypeStruct((N, wenc.shape[1]), _F32),
        jax.ShapeDtypeStruct((N, wdec.shape[1]), _F32),
    ]
    return pl.pallas_call(_emb_kernel, out_shape=out_shape)(
        adj, cell, wenc, wdec)


def _dec_kernel(adj_ref, xr_ref, xc_ref, dw1_ref, dw2_ref, dw3_ref,
                ew1_ref, ew2_ref, ew3_ref, recon_ref, cross_ref):
    a = adj_ref[...]

    def hop(x, w_ref):
        return _dot(a, _dot(x, w_ref[...]).astype(_BF16))

    r = _relu(hop(xr_ref[...].astype(_F32), dw1_ref))
    r = _relu(hop(r, dw2_ref))
    recon_ref[...] = hop(r, dw3_ref)
    c = _relu(hop(xc_ref[...].astype(_F32), dw1_ref))
    c = _relu(hop(c, dw2_ref))
    c = hop(c, dw3_ref)
    c = _relu(hop(c, ew1_ref))
    c = _relu(hop(c, ew2_ref))
    cross_ref[...] = hop(c, ew3_ref)


def _dec_chain(adj16, x_recon, x_cross, dw1, dw2, dw3, ew1, ew2, ew3):
    out_shape = [
        jax.ShapeDtypeStruct((N, dw3.shape[1]), _F32),
        jax.ShapeDtypeStruct((N, ew3.shape[1]), _F32),
    ]
    return pl.pallas_call(_dec_kernel, out_shape=out_shape)(
        adj16, x_recon, x_cross, dw1, dw2, dw3, ew1, ew2, ew3)


def _att_pair(e1, e2, w_ref, u_ref):
    u_row = jnp.reshape(u_ref[...], (1, -1))
    v1 = jnp.tanh(_dot(e1, w_ref[...]))
    v2 = jnp.tanh(_dot(e2, w_ref[...]))
    t1 = jnp.sum(v1 * u_row, axis=1, keepdims=True)
    t2 = jnp.sum(v2 * u_row, axis=1, keepdims=True)
    m = jnp.maximum(t1, t2)
    z1 = jnp.exp(t1 - m)
    z2 = jnp.exp(t2 - m)
    s = z1 + z2
    a1 = z1 / s
    a2 = z2 / s
    comb = a1 * e1 + a2 * e2
    alpha = jnp.concatenate([a1, a2], axis=1)
    return comb, alpha


def _att_kernel(spa_ref, ls1_ref, fea_ref, lf1_ref, ls2_ref, lf2_ref,
                w1_ref, u1_ref, w2_ref, u2_ref, wf_ref, uf_ref,
                wo2_ref, uo2_ref, wc_ref, uc_ref,
                o1_ref, o2_ref, d_ref, a1_ref, a2_ref, af_ref, ao2_ref,
                ac_ref):
    att1, alpha1 = _att_pair(spa_ref[...], ls1_ref[...], w1_ref, u1_ref)
    att2, alpha2 = _att_pair(fea_ref[...], lf1_ref[...], w2_ref, u2_ref)
    o1, alphaf = _att_pair(att1, att2, wf_ref, uf_ref)
    o2, alphao2 = _att_pair(ls2_ref[...], lf2_ref[...], wo2_ref, uo2_ref)
    d, alphac = _att_pair(o1, o2, wc_ref, uc_ref)
    o1_ref[...] = o1
    o2_ref[...] = o2
    d_ref[...] = d
    a1_ref[...] = alpha1
    a2_ref[...] = alpha2
    af_ref[...] = alphaf
    ao2_ref[...] = alphao2
    ac_ref[...] = alphac


def _attention(emb_spa, latent_spa1, emb_fea, latent_fea1, latent_spa2,
               latent_fea2, att1_w, att1_u, att2_w, att2_u, attf_w, attf_u,
               atto2_w, atto2_u, attc_w, attc_u):
    d = emb_spa.shape[1]
    out_shape = [
        jax.ShapeDtypeStruct((N, d), _F32),
        jax.ShapeDtypeStruct((N, d), _F32),
        jax.ShapeDtypeStruct((N, d), _F32),
        jax.ShapeDtypeStruct((N, 2), _F32),
        jax.ShapeDtypeStruct((N, 2), _F32),
        jax.ShapeDtypeStruct((N, 2), _F32),
        jax.ShapeDtypeStruct((N, 2), _F32),
        jax.ShapeDtypeStruct((N, 2), _F32),
    ]
    return pl.pallas_call(_att_kernel, out_shape=out_shape)(
        emb_spa, latent_spa1, emb_fea, latent_fea1, latent_spa2, latent_fea2,
        att1_w, att1_u, att2_w, att2_u, attf_w, attf_u, atto2_w, atto2_u,
        attc_w, attc_u)


def kernel(f_omics1, f_omics2, adj_spa1, adj_fea1, adj_spa2, adj_fea2,
           cell_emb, adj_emb, W_emb_enc, W_emb_dec,
           enc1_W1, enc1_W2, enc1_W3, dec1_W1, dec1_W2, dec1_W3,
           enc2_W1, enc2_W2, enc2_W3, dec2_W1, dec2_W2, dec2_W3,
           att1_w, att1_u, att2_w, att2_u, attf_w, attf_u,
           atto2_w, atto2_u, attc_w, attc_u):
    spa1_16 = adj_spa1.astype(_BF16)
    spa2_16 = adj_spa2.astype(_BF16)

    emb_spa, latent_spa1, recon_spa = _enc_spa1(
        adj_spa1, cell_emb, W_emb_enc, W_emb_dec, f_omics1,
        enc1_W1, enc1_W2, enc1_W3)
    latent_fea1 = _enc_plain(adj_fea1, f_omics1, enc1_W1, enc1_W2, enc1_W3)
    latent_spa2 = _enc_plain(adj_spa2, f_omics2, enc2_W1, enc2_W2, enc2_W3)
    latent_fea2 = _enc_plain(adj_fea2, f_omics2, enc2_W1, enc2_W2, enc2_W3)
    emb_fea, recon_fea = _emb_chain(adj_emb, cell_emb, W_emb_enc, W_emb_dec)

    (o1, o2, comb, alpha_att1, alpha_att2, alpha_att_omics1, alpha_omics2,
     alpha) = _attention(
        emb_spa, latent_spa1, emb_fea, latent_fea1, latent_spa2, latent_fea2,
        att1_w, att1_u, att2_w, att2_u, attf_w, attf_u, atto2_w, atto2_u,
        attc_w, attc_u)

    emb_recon1, emb_cross2 = _dec_chain(
        spa1_16, comb, o2, dec1_W1, dec1_W2, dec1_W3,
        enc1_W1, enc1_W2, enc1_W3)
    emb_recon2, emb_cross1 = _dec_chain(
        spa2_16, comb, o1, dec2_W1, dec2_W2, dec2_W3,
        enc2_W1, enc2_W2, enc2_W3)

    return (o1, o2, comb, emb_recon1, emb_recon2, emb_cross1, emb_cross2,
            alpha_att1, alpha_att2, alpha_att_omics1, alpha_omics2, alpha,
            recon_spa, recon_fea)
